# Initial kernel scaffold; baseline (speedup 1.0000x reference)
import jax
import jax.numpy as jnp
from jax.experimental import pallas as pl


def kernel(x, token_table, pos_embedding, segment_table):
    raise NotImplementedError("write your pallas kernel here")


# speedup vs baseline: 2.8816x; 2.8816x over previous
import functools

import jax
import jax.numpy as jnp
from jax import lax
from jax.experimental import pallas as pl
from jax.experimental.pallas import tpu as pltpu
from jax.experimental.pallas import tpu_sc as plsc

L_SEQ = 2048
N_BATCH = 4
D_MODEL = 768
LANES = 16
DV = D_MODEL // LANES

NC, NS = 2, 16
NW = NC * NS
L_PER_W = L_SEQ // NW
CH = 8
NCH = L_PER_W // CH


def _body(x_hbm, tok_hbm, pos_hbm, seg_hbm, out_hbm,
          x_v, tok_v, seg_v, c0_v, diff_v, pos_v, out_v, sem):
    wid = lax.axis_index("s") * NC + lax.axis_index("c")
    base = wid * L_PER_W

    pltpu.sync_copy(x_hbm.at[pl.ds(base * N_BATCH, L_PER_W * N_BATCH)], x_v)
    pltpu.sync_copy(tok_hbm.at[pl.ds(0, 2)], tok_v)
    pltpu.sync_copy(seg_hbm.at[pl.ds(0, 2)], seg_v)

    for j in range(DV):
        sl = pl.ds(j * LANES, LANES)
        t0 = tok_v[0, sl] + seg_v[0, sl]
        t1 = tok_v[1, sl] + seg_v[1, sl]
        c0_v[sl] = t0
        diff_v[sl] = t1 - t0

    def chunk_body(c, _):
        lo = base + c * CH
        pltpu.sync_copy(pos_hbm.at[pl.ds(lo, CH)], pos_v)

        def g_body(g, _):
            xvec = x_v[pl.ds(c * (CH * N_BATCH) + g * LANES, LANES)]
            for ii in range(4):
                i = g * 4 + ii
                spl = [jnp.full((LANES,), xvec[ii * N_BATCH + n], jnp.float32)
                       for n in range(N_BATCH)]
                for j in range(DV):
                    sl = pl.ds(j * LANES, LANES)
                    t = pos_v[i, sl] + c0_v[sl]
                    df = diff_v[sl]
                    for n in range(N_BATCH):
                        out_v[i, n, sl] = t + spl[n] * df
            return 0

        lax.fori_loop(0, CH // 4, g_body, 0)
        pltpu.sync_copy(out_v, out_hbm.at[pl.ds(lo, CH)])
        return 0

    lax.fori_loop(0, NCH, chunk_body, 0)


@jax.jit
def _sc_embed(xt, token_table, pos2d, segment_table):
    mesh = plsc.VectorSubcoreMesh(core_axis_name="c", subcore_axis_name="s")
    kfn = pl.kernel(
        _body,
        out_type=jax.ShapeDtypeStruct((L_SEQ, N_BATCH, D_MODEL), jnp.float32),
        mesh=mesh,
        scratch_types=[
            pltpu.VMEM((L_PER_W * N_BATCH,), jnp.float32),
            pltpu.VMEM((2, D_MODEL), jnp.float32),
            pltpu.VMEM((2, D_MODEL), jnp.float32),
            pltpu.VMEM((D_MODEL,), jnp.float32),
            pltpu.VMEM((D_MODEL,), jnp.float32),
            pltpu.VMEM((CH, D_MODEL), jnp.float32),
            pltpu.VMEM((CH, N_BATCH, D_MODEL), jnp.float32),
            pltpu.SemaphoreType.DMA,
        ],
    )
    return kfn(xt, token_table, pos2d, segment_table)


def kernel(x, token_table, pos_embedding, segment_table):
    xt = jnp.transpose(x, (1, 0)).astype(jnp.float32).reshape(-1)
    pos2d = pos_embedding.reshape(pos_embedding.shape[0], D_MODEL)[:L_SEQ]
    return _sc_embed(xt, token_table, pos2d, segment_table)


# speedup vs baseline: 3.6046x; 1.2509x over previous
import functools

import jax
import jax.numpy as jnp
from jax import lax
from jax.experimental import pallas as pl
from jax.experimental.pallas import tpu as pltpu
from jax.experimental.pallas import tpu_sc as plsc

L_SEQ = 2048
N_BATCH = 4
D_MODEL = 768
LANES = 16
DV = D_MODEL // LANES

NC, NS = 2, 16
NW = NC * NS
L_PER_W = L_SEQ // NW
CH = 16
NCH = L_PER_W // CH
NGR = CH // 4


def _body(x_hbm, tok_hbm, pos_hbm, seg_hbm, out_hbm,
          x_v, tok_v, seg_v, c0_v, diff_v, pos_v, out_v, sem_pos, sem_out):
    wid = lax.axis_index("s") * NC + lax.axis_index("c")
    base = wid * L_PER_W

    pltpu.sync_copy(x_hbm.at[pl.ds(base * N_BATCH, L_PER_W * N_BATCH)], x_v)
    pltpu.sync_copy(tok_hbm.at[pl.ds(0, 2)], tok_v)
    pltpu.sync_copy(seg_hbm.at[pl.ds(0, 2)], seg_v)

    for j in range(DV):
        sl = pl.ds(j * LANES, LANES)
        t0 = tok_v[0, sl] + seg_v[0, sl]
        t1 = tok_v[1, sl] + seg_v[1, sl]
        c0_v[sl] = t0
        diff_v[sl] = t1 - t0

    def pos_copy(c, b):
        return pltpu.make_async_copy(
            pos_hbm.at[pl.ds(base + c * CH, CH)], pos_v.at[b], sem_pos.at[b])

    def out_copy(c, b):
        return pltpu.make_async_copy(
            out_v.at[b], out_hbm.at[pl.ds(base + c * CH, CH)], sem_out.at[b])

    pos_copy(0, 0).start()

    def chunk_body(c, _):
        b = lax.rem(c, 2)
        nxt = lax.rem(c + 1, 2)

        @pl.when(c + 1 < NCH)
        def _():
            pos_copy(c + 1, nxt).start()

        pos_copy(c, b).wait()

        @pl.when(c >= 2)
        def _():
            out_copy(c - 2, b).wait()

        def g_body(g, _):
            xvec = x_v[pl.ds(c * (CH * N_BATCH) + g * LANES, LANES)]
            spl = [[jnp.full((LANES,), xvec[ii * N_BATCH + n], jnp.float32)
                    for n in range(N_BATCH)] for ii in range(4)]
            for j in range(DV):
                sl = pl.ds(j * LANES, LANES)
                c0j = c0_v[sl]
                dfj = diff_v[sl]
                for ii in range(4):
                    i = g * 4 + ii
                    t = pos_v[b, i, sl] + c0j
                    for n in range(N_BATCH):
                        out_v[b, i, n, sl] = t + spl[ii][n] * dfj
            return 0

        lax.fori_loop(0, NGR, g_body, 0)
        out_copy(c, b).start()
        return 0

    lax.fori_loop(0, NCH, chunk_body, 0)

    for c in range(max(NCH - 2, 0), NCH):
        out_copy(c, c % 2).wait()


@jax.jit
def _sc_embed(xt, token_table, pos2d, segment_table):
    mesh = plsc.VectorSubcoreMesh(core_axis_name="c", subcore_axis_name="s")
    kfn = pl.kernel(
        _body,
        out_type=jax.ShapeDtypeStruct((L_SEQ, N_BATCH, D_MODEL), jnp.float32),
        mesh=mesh,
        scratch_types=[
            pltpu.VMEM((L_PER_W * N_BATCH,), jnp.float32),
            pltpu.VMEM((2, D_MODEL), jnp.float32),
            pltpu.VMEM((2, D_MODEL), jnp.float32),
            pltpu.VMEM((D_MODEL,), jnp.float32),
            pltpu.VMEM((D_MODEL,), jnp.float32),
            pltpu.VMEM((2, CH, D_MODEL), jnp.float32),
            pltpu.VMEM((2, CH, N_BATCH, D_MODEL), jnp.float32),
            pltpu.SemaphoreType.DMA((2,)),
            pltpu.SemaphoreType.DMA((2,)),
        ],
    )
    return kfn(xt, token_table, pos2d, segment_table)


def kernel(x, token_table, pos_embedding, segment_table):
    xt = jnp.transpose(x, (1, 0)).astype(jnp.float32).reshape(-1)
    pos2d = pos_embedding.reshape(pos_embedding.shape[0], D_MODEL)[:L_SEQ]
    return _sc_embed(xt, token_table, pos2d, segment_table)
